# Initial kernel scaffold; baseline (speedup 1.0000x reference)
import jax
import jax.numpy as jnp
from jax.experimental import pallas as pl


def kernel(edge_features, L1, W1, b1, W2, b2, W3, b3):
    raise NotImplementedError("write your pallas kernel here")


# speedup vs baseline: 3.9605x; 3.9605x over previous
import jax
import jax.numpy as jnp
from jax.experimental import pallas as pl

N = 512
F = 128
H1 = 64
H2 = 32
D = 4
B = 128
G = N // B
EPS = 1e-4


def _s0(eft_ref, w1at_ref, w1bt_ref, b1_ref, at_ref, bt_ref):
    eft = eft_ref[...]
    at_ref[...] = (
        jnp.dot(w1at_ref[...], eft, preferred_element_type=jnp.float32) + b1_ref[...]
    )
    bt_ref[...] = jnp.dot(w1bt_ref[...], eft, preferred_element_type=jnp.float32)


def _s1(at_ref, bt_ref, l1_ref, w2t_ref, b2_ref, w3t_ref, b3_ref, pt_ref, diag_ref):
    i = pl.program_id(0)
    j = pl.program_id(1)
    a = at_ref[...]
    b = bt_ref[...]
    h1 = jnp.maximum(a[:, :, None] + b[:, None, :], 0.0).reshape(H1, B * B)
    h2 = jnp.maximum(
        jnp.dot(w2t_ref[...], h1, preferred_element_type=jnp.float32) + b2_ref[...],
        0.0,
    )
    pt = jnp.dot(w3t_ref[...], h2, preferred_element_type=jnp.float32) + b3_ref[...]
    pt3 = pt.reshape(D, B, B)
    pt_ref[...] = pt3
    l1 = l1_ref[...]
    ri = jax.lax.broadcasted_iota(jnp.int32, (B, B), 0) + i * B
    ci = jax.lax.broadcasted_iota(jnp.int32, (B, B), 1) + j * B
    m = jnp.where((jnp.abs(l1) > 0.0) & (ri != ci), 1.0, 0.0)
    contrib = jnp.sum(pt3 * pt3 * m[None], axis=2)

    @pl.when(j == 0)
    def _():
        diag_ref[...] = contrib

    @pl.when(j != 0)
    def _():
        diag_ref[...] += contrib


def _s2(pt_ref, ptr_ref, l1_ref, l1r_ref, di_ref, dj_ref, out_ref):
    i = pl.program_id(0)
    j = pl.program_id(1)
    p_ij = pt_ref[...]
    p_ji = jnp.swapaxes(ptr_ref[...], 1, 2)
    l1 = l1_ref[...]
    l1r = jnp.swapaxes(l1r_ref[...], 0, 1)
    coef = -jnp.sign(l1) * jnp.where(jnp.abs(l1r) > 0.0, 1.0, 0.0)
    off = coef[None] * p_ji * p_ij
    diag_i = di_ref[...]
    diag_j = dj_ref[...]
    dinv_i = 1.0 / jnp.sqrt(diag_i + EPS)
    dinv_j = 1.0 / jnp.sqrt(diag_j + EPS)
    ri = jax.lax.broadcasted_iota(jnp.int32, (B, B), 0) + i * B
    ci = jax.lax.broadcasted_iota(jnp.int32, (B, B), 1) + j * B
    dvals = jnp.where((ri == ci)[None], diag_i[:, :, None], off)
    s = dvals * dinv_i[:, :, None] * dinv_j[:, None, :]
    z = jnp.zeros((B, B), jnp.float32)
    for aa in range(D):
        for bb in range(D):
            out_ref[aa, bb] = s[aa] if aa == bb else z


def kernel(edge_features, L1, W1, b1, W2, b2, W3, b3):
    eft = edge_features.T
    w1at = W1[:F].T
    w1bt = W1[F:].T
    b1c = b1.reshape(H1, 1)
    w2t = W2.T
    b2c = b2.reshape(H2, 1)
    w3t = W3.T
    b3c = b3.reshape(D, 1)

    at, bt = pl.pallas_call(
        _s0,
        out_shape=[
            jax.ShapeDtypeStruct((H1, N), jnp.float32),
            jax.ShapeDtypeStruct((H1, N), jnp.float32),
        ],
    )(eft, w1at, w1bt, b1c)

    pt, diag = pl.pallas_call(
        _s1,
        grid=(G, G),
        in_specs=[
            pl.BlockSpec((H1, B), lambda i, j: (0, i)),
            pl.BlockSpec((H1, B), lambda i, j: (0, j)),
            pl.BlockSpec((B, B), lambda i, j: (i, j)),
            pl.BlockSpec((H2, H1), lambda i, j: (0, 0)),
            pl.BlockSpec((H2, 1), lambda i, j: (0, 0)),
            pl.BlockSpec((D, H2), lambda i, j: (0, 0)),
            pl.BlockSpec((D, 1), lambda i, j: (0, 0)),
        ],
        out_specs=[
            pl.BlockSpec((D, B, B), lambda i, j: (0, i, j)),
            pl.BlockSpec((D, B), lambda i, j: (0, i)),
        ],
        out_shape=[
            jax.ShapeDtypeStruct((D, N, N), jnp.float32),
            jax.ShapeDtypeStruct((D, N), jnp.float32),
        ],
    )(at, bt, L1, w2t, b2c, w3t, b3c)

    out6 = pl.pallas_call(
        _s2,
        grid=(G, G),
        in_specs=[
            pl.BlockSpec((D, B, B), lambda i, j: (0, i, j)),
            pl.BlockSpec((D, B, B), lambda i, j: (0, j, i)),
            pl.BlockSpec((B, B), lambda i, j: (i, j)),
            pl.BlockSpec((B, B), lambda i, j: (j, i)),
            pl.BlockSpec((D, B), lambda i, j: (0, i)),
            pl.BlockSpec((D, B), lambda i, j: (0, j)),
        ],
        out_specs=pl.BlockSpec((D, D, B, B), lambda i, j: (0, 0, i, j)),
        out_shape=jax.ShapeDtypeStruct((D, D, N, N), jnp.float32),
    )(pt, pt, L1, L1, diag, diag)

    return out6.transpose(2, 0, 3, 1).reshape(N * D, N * D)


# speedup vs baseline: 87.4234x; 22.0736x over previous
import jax
import jax.numpy as jnp
from jax.experimental import pallas as pl

N = 512
F = 128
H1 = 64
H2 = 32
D = 4
B = 128
G = N // B
EPS = 1e-4


def _s0(eft_ref, w1at_ref, w1bt_ref, b1_ref, at_ref, bt_ref):
    eft = eft_ref[...]
    at_ref[...] = (
        jnp.dot(w1at_ref[...], eft, preferred_element_type=jnp.float32) + b1_ref[...]
    )
    bt_ref[...] = jnp.dot(w1bt_ref[...], eft, preferred_element_type=jnp.float32)


def _s1(at_ref, bt_ref, l1_ref, w2t_ref, b2_ref, w3t_ref, b3_ref, pt_ref, diag_ref):
    i = pl.program_id(0)
    j = pl.program_id(1)
    a = at_ref[...]
    b = bt_ref[...]
    h1 = jnp.maximum(a[:, :, None] + b[:, None, :], 0.0).reshape(H1, B * B)
    h2 = jnp.maximum(
        jnp.dot(w2t_ref[...], h1, preferred_element_type=jnp.float32) + b2_ref[...],
        0.0,
    )
    pt = jnp.dot(w3t_ref[...], h2, preferred_element_type=jnp.float32) + b3_ref[...]
    pt3 = pt.reshape(D, B, B)
    pt_ref[...] = pt3
    l1 = l1_ref[...]
    ri = jax.lax.broadcasted_iota(jnp.int32, (B, B), 0) + i * B
    ci = jax.lax.broadcasted_iota(jnp.int32, (B, B), 1) + j * B
    m = jnp.where((jnp.abs(l1) > 0.0) & (ri != ci), 1.0, 0.0)
    contrib = jnp.sum(pt3 * pt3 * m[None], axis=2)

    @pl.when(j == 0)
    def _():
        diag_ref[...] = contrib

    @pl.when(j != 0)
    def _():
        diag_ref[...] += contrib


def _s2(pt_ref, ptr_ref, l1_ref, l1r_ref, di_ref, dj_ref, out_ref):
    i = pl.program_id(0)
    j = pl.program_id(1)
    p_ij = pt_ref[...]
    p_ji = jnp.swapaxes(ptr_ref[...], 1, 2)
    l1 = l1_ref[...]
    l1r = jnp.swapaxes(l1r_ref[...], 0, 1)
    coef = -jnp.sign(l1) * jnp.where(jnp.abs(l1r) > 0.0, 1.0, 0.0)
    off = coef[None] * p_ji * p_ij
    diag_i = di_ref[...]
    diag_j = dj_ref[...]
    dinv_i = 1.0 / jnp.sqrt(diag_i + EPS)
    dinv_j = 1.0 / jnp.sqrt(diag_j + EPS)
    ri = jax.lax.broadcasted_iota(jnp.int32, (B, B), 0) + i * B
    ci = jax.lax.broadcasted_iota(jnp.int32, (B, B), 1) + j * B
    dvals = jnp.where((ri == ci)[None], diag_i[:, :, None], off)
    s = dvals * dinv_i[:, :, None] * dinv_j[:, None, :]
    gr = jax.lax.broadcasted_iota(jnp.int32, (D * B, B), 0)
    gc = jax.lax.broadcasted_iota(jnp.int32, (D * B, B), 1)
    tr = jax.lax.broadcasted_iota(jnp.int32, (B, D * B), 0)
    tc = jax.lax.broadcasted_iota(jnp.int32, (B, D * B), 1)
    acc = jnp.zeros((D * B, D * B), jnp.float32)
    for aa in range(D):
        g = jnp.where(gr == D * gc + aa, 1.0, 0.0)
        gt = jnp.where(tc == D * tr + aa, 1.0, 0.0)
        t = jnp.dot(s[aa], gt, preferred_element_type=jnp.float32)
        acc = acc + jnp.dot(g, t, preferred_element_type=jnp.float32)
    out_ref[...] = acc


def kernel(edge_features, L1, W1, b1, W2, b2, W3, b3):
    eft = edge_features.T
    w1at = W1[:F].T
    w1bt = W1[F:].T
    b1c = b1.reshape(H1, 1)
    w2t = W2.T
    b2c = b2.reshape(H2, 1)
    w3t = W3.T
    b3c = b3.reshape(D, 1)

    at, bt = pl.pallas_call(
        _s0,
        out_shape=[
            jax.ShapeDtypeStruct((H1, N), jnp.float32),
            jax.ShapeDtypeStruct((H1, N), jnp.float32),
        ],
    )(eft, w1at, w1bt, b1c)

    pt, diag = pl.pallas_call(
        _s1,
        grid=(G, G),
        in_specs=[
            pl.BlockSpec((H1, B), lambda i, j: (0, i)),
            pl.BlockSpec((H1, B), lambda i, j: (0, j)),
            pl.BlockSpec((B, B), lambda i, j: (i, j)),
            pl.BlockSpec((H2, H1), lambda i, j: (0, 0)),
            pl.BlockSpec((H2, 1), lambda i, j: (0, 0)),
            pl.BlockSpec((D, H2), lambda i, j: (0, 0)),
            pl.BlockSpec((D, 1), lambda i, j: (0, 0)),
        ],
        out_specs=[
            pl.BlockSpec((D, B, B), lambda i, j: (0, i, j)),
            pl.BlockSpec((D, B), lambda i, j: (0, i)),
        ],
        out_shape=[
            jax.ShapeDtypeStruct((D, N, N), jnp.float32),
            jax.ShapeDtypeStruct((D, N), jnp.float32),
        ],
    )(at, bt, L1, w2t, b2c, w3t, b3c)

    out6 = pl.pallas_call(
        _s2,
        grid=(G, G),
        in_specs=[
            pl.BlockSpec((D, B, B), lambda i, j: (0, i, j)),
            pl.BlockSpec((D, B, B), lambda i, j: (0, j, i)),
            pl.BlockSpec((B, B), lambda i, j: (i, j)),
            pl.BlockSpec((B, B), lambda i, j: (j, i)),
            pl.BlockSpec((D, B), lambda i, j: (0, i)),
            pl.BlockSpec((D, B), lambda i, j: (0, j)),
        ],
        out_specs=pl.BlockSpec((D * B, D * B), lambda i, j: (i, j)),
        out_shape=jax.ShapeDtypeStruct((N * D, N * D), jnp.float32),
    )(pt, pt, L1, L1, diag, diag)

    return out6


# speedup vs baseline: 109.0481x; 1.2474x over previous
import numpy as np
import jax
import jax.numpy as jnp
from jax.experimental import pallas as pl

N = 512
F = 128
H1 = 64
H2 = 32
D = 4
B = 128
G = N // B
BI = 32
GI = N // BI
EPS = 1e-4

_EROW = np.kron(np.eye(BI, dtype=np.float32), np.ones((1, N), dtype=np.float32))
_GCT = np.zeros((D, B, D * B), dtype=np.float32)
for _a in range(D):
    _GCT[_a, np.arange(B), D * np.arange(B) + _a] = 1.0
_GPERM = np.zeros((D * B, D * B), dtype=np.float32)
for _a in range(D):
    _GPERM[D * np.arange(B) + _a, _a * B + np.arange(B)] = 1.0


def _s0(ef_ref, eft_ref, w1a_ref, w1bt_ref, b1r_ref, att_ref, bt_ref):
    att_ref[...] = (
        jnp.dot(ef_ref[...], w1a_ref[...], preferred_element_type=jnp.float32)
        + b1r_ref[...]
    )
    bt_ref[...] = jnp.dot(
        w1bt_ref[...], eft_ref[...], preferred_element_type=jnp.float32
    )


def _s1(at_ref, bt_ref, l1_ref, w2t_ref, b2_ref, w3t_ref, b3_ref, erow_ref, pt_ref, diag_ref):
    ib = pl.program_id(0)
    a_blk_t = at_ref[...]
    bt = bt_ref[...]
    a_exp = jax.lax.dot_general(
        a_blk_t,
        erow_ref[...],
        (((0,), (0,)), ((), ())),
        preferred_element_type=jnp.float32,
    )
    b_exp = jnp.concatenate([bt] * BI, axis=1)
    h1 = jnp.maximum(a_exp + b_exp, 0.0)
    h2 = jnp.maximum(
        jnp.dot(w2t_ref[...], h1, preferred_element_type=jnp.float32) + b2_ref[...],
        0.0,
    )
    pt = jnp.dot(w3t_ref[...], h2, preferred_element_type=jnp.float32) + b3_ref[...]
    pt3 = pt.reshape(D, BI, N)
    pt_ref[...] = pt3
    l1 = l1_ref[...]
    ri = jax.lax.broadcasted_iota(jnp.int32, (BI, N), 0) + ib * BI
    ci = jax.lax.broadcasted_iota(jnp.int32, (BI, N), 1)
    m = jnp.where((jnp.abs(l1) > 0.0) & (ri != ci), 1.0, 0.0)
    contrib = jnp.sum(pt3 * pt3 * m[None], axis=2)
    diag_ref[...] = jnp.swapaxes(contrib, 0, 1)


def _s2(pt_ref, ptr_ref, l1_ref, l1r_ref, di_ref, dj_ref, gct_ref, gperm_ref, out_ref):
    i = pl.program_id(0)
    j = pl.program_id(1)
    p_ij = pt_ref[...]
    p_ji = jnp.swapaxes(ptr_ref[...], 1, 2)
    l1 = l1_ref[...]
    l1r = jnp.swapaxes(l1r_ref[...], 0, 1)
    coef = -jnp.sign(l1) * jnp.where(jnp.abs(l1r) > 0.0, 1.0, 0.0)
    off = coef[None] * p_ji * p_ij
    diag_i = jnp.swapaxes(di_ref[...], 0, 1)
    diag_j = jnp.swapaxes(dj_ref[...], 0, 1)
    dinv_i = 1.0 / jnp.sqrt(diag_i + EPS)
    dinv_j = 1.0 / jnp.sqrt(diag_j + EPS)
    ri = jax.lax.broadcasted_iota(jnp.int32, (B, B), 0) + i * B
    ci = jax.lax.broadcasted_iota(jnp.int32, (B, B), 1) + j * B
    dvals = jnp.where((ri == ci)[None], diag_i[:, :, None], off)
    s = dvals * dinv_i[:, :, None] * dinv_j[:, None, :]
    gct = gct_ref[...]
    tcat = jnp.concatenate(
        [
            jnp.dot(s[aa], gct[aa], preferred_element_type=jnp.float32)
            for aa in range(D)
        ],
        axis=0,
    )
    out_ref[...] = jnp.dot(gperm_ref[...], tcat, preferred_element_type=jnp.float32)


def kernel(edge_features, L1, W1, b1, W2, b2, W3, b3):
    eft = edge_features.T
    w1bt = W1[F:].T
    w2t = W2.T
    b2c = b2.reshape(H2, 1)
    w3t = W3.T
    b3c = b3.reshape(D, 1)

    at, bt = pl.pallas_call(
        _s0,
        out_shape=[
            jax.ShapeDtypeStruct((N, H1), jnp.float32),
            jax.ShapeDtypeStruct((H1, N), jnp.float32),
        ],
    )(edge_features, eft, W1[:F], w1bt, b1.reshape(1, H1))

    pt, diag = pl.pallas_call(
        _s1,
        grid=(GI,),
        in_specs=[
            pl.BlockSpec((BI, H1), lambda i: (i, 0)),
            pl.BlockSpec((H1, N), lambda i: (0, 0)),
            pl.BlockSpec((BI, N), lambda i: (i, 0)),
            pl.BlockSpec((H2, H1), lambda i: (0, 0)),
            pl.BlockSpec((H2, 1), lambda i: (0, 0)),
            pl.BlockSpec((D, H2), lambda i: (0, 0)),
            pl.BlockSpec((D, 1), lambda i: (0, 0)),
            pl.BlockSpec((BI, BI * N), lambda i: (0, 0)),
        ],
        out_specs=[
            pl.BlockSpec((D, BI, N), lambda i: (0, i, 0)),
            pl.BlockSpec((BI, D), lambda i: (i, 0)),
        ],
        out_shape=[
            jax.ShapeDtypeStruct((D, N, N), jnp.float32),
            jax.ShapeDtypeStruct((N, D), jnp.float32),
        ],
    )(at, bt, L1, w2t, b2c, w3t, b3c, jnp.asarray(_EROW))

    out6 = pl.pallas_call(
        _s2,
        grid=(G, G),
        in_specs=[
            pl.BlockSpec((D, B, B), lambda i, j: (0, i, j)),
            pl.BlockSpec((D, B, B), lambda i, j: (0, j, i)),
            pl.BlockSpec((B, B), lambda i, j: (i, j)),
            pl.BlockSpec((B, B), lambda i, j: (j, i)),
            pl.BlockSpec((B, D), lambda i, j: (i, 0)),
            pl.BlockSpec((B, D), lambda i, j: (j, 0)),
            pl.BlockSpec((D, B, D * B), lambda i, j: (0, 0, 0)),
            pl.BlockSpec((D * B, D * B), lambda i, j: (0, 0)),
        ],
        out_specs=pl.BlockSpec((D * B, D * B), lambda i, j: (i, j)),
        out_shape=jax.ShapeDtypeStruct((N * D, N * D), jnp.float32),
    )(pt, pt, L1, L1, diag, diag, jnp.asarray(_GCT), jnp.asarray(_GPERM))

    return out6
